# baseline (speedup 1.0000x reference)
import functools

import jax
import jax.numpy as jnp
from jax import lax
from jax.experimental import pallas as pl
from jax.experimental.pallas import tpu as pltpu
from jax.experimental.pallas import tpu_sc as plsc

_NBUF = 2
_NW = 32


def _gather_rows(b, s, d):
    q_per_w = b // _NW
    mesh = plsc.VectorSubcoreMesh(core_axis_name="c", subcore_axis_name="s")

    @functools.partial(
        pl.kernel,
        mesh=mesh,
        compiler_params=pltpu.CompilerParams(use_tc_tiling_on_sc=False),
        out_type=jax.ShapeDtypeStruct((b, s, d), jnp.float32),
        scratch_types=[
            pltpu.VMEM((q_per_w, s), jnp.int32),
            [pltpu.VMEM((s, d), jnp.float32)] * _NBUF,
            [pltpu.SemaphoreType.DMA] * _NBUF,
            [pltpu.SemaphoreType.DMA] * _NBUF,
        ],
    )
    def k(idx_hbm, table_hbm, out_hbm, idx_v, bufs, gsems, ssems):
        nc = lax.axis_size("c")
        wid = lax.axis_index("s") * nc + lax.axis_index("c")
        q0 = wid * q_per_w
        pltpu.sync_copy(idx_hbm.at[pl.ds(q0, q_per_w)], idx_v)

        def gather(i, bf):
            pltpu.async_copy(
                table_hbm.at[idx_v.at[i]], bufs[bf], gsems[bf]
            )

        def scatter(i, bf):
            pltpu.async_copy(bufs[bf], out_hbm.at[q0 + i], ssems[bf])

        def gwait(bf):
            pltpu.make_async_copy(
                table_hbm.at[pl.ds(0, s)], bufs[bf], gsems[bf]
            ).wait()

        def swait(bf):
            pltpu.make_async_copy(bufs[bf], out_hbm.at[q0], ssems[bf]).wait()

        for bf in range(_NBUF):
            gather(bf, bf)

        @pl.loop(_NBUF, q_per_w, step=_NBUF)
        def body(i0):
            for bf in range(_NBUF):
                gwait(bf)
                scatter(i0 - _NBUF + bf, bf)
            for bf in range(_NBUF):
                swait(bf)
                gather(i0 + bf, bf)

        for bf in range(_NBUF):
            gwait(bf)
            scatter(q_per_w - _NBUF + bf, bf)
        for bf in range(_NBUF):
            swait(bf)

    return k


def kernel(x, embedding):
    b, s = x.shape
    v, d = embedding.shape
    return _gather_rows(b, s, d)(x.astype(jnp.int32), embedding)
